# Initial kernel scaffold; baseline (speedup 1.0000x reference)
import jax
import jax.numpy as jnp
from jax.experimental import pallas as pl


def kernel(x, conv1_w, conv1_b, conv2_w, conv2_b, fc1_w, fc1_b, fc2_w, fc2_b):
    raise NotImplementedError("write your pallas kernel here")


# speedup vs baseline: 26.3895x; 26.3895x over previous
import numpy as np

import jax
import jax.numpy as jnp
from jax.experimental import pallas as pl
from jax.experimental.pallas import tpu as pltpu

_TB = 32


def _fused_kernel(x_ref, b1_ref, b2_ref, f1_ref, f2_ref,
                  bb1_ref, bb2_ref, fb1_ref, fb2_ref, o_ref):
    tb = x_ref.shape[0]
    xp = jnp.pad(x_ref[...], ((0, 0), (2, 2), (2, 2))).astype(jnp.bfloat16)

    acc_e = acc_o = None
    for kh in range(5):
        a = xp[:, kh:kh + 28, :].reshape(tb * 28, 32)
        pe = jnp.dot(a, b1_ref[2 * kh], preferred_element_type=jnp.float32)
        po = jnp.dot(a, b1_ref[2 * kh + 1], preferred_element_type=jnp.float32)
        acc_e = pe if acc_e is None else acc_e + pe
        acc_o = po if acc_o is None else acc_o + po
    m = jnp.maximum(jnp.maximum(acc_e, acc_o) + bb1_ref[...], 0.0)
    m = m.reshape(tb, 14, 2, 448)
    hp = jnp.maximum(m[:, :, 0, :], m[:, :, 1, :]).astype(jnp.bfloat16)

    acc_e = acc_o = None
    for kh in range(5):
        a = hp[:, kh:kh + 10, :].reshape(tb * 10, 448)
        pe = jnp.dot(a, b2_ref[2 * kh], preferred_element_type=jnp.float32)
        po = jnp.dot(a, b2_ref[2 * kh + 1], preferred_element_type=jnp.float32)
        acc_e = pe if acc_e is None else acc_e + pe
        acc_o = po if acc_o is None else acc_o + po
    m2 = jnp.maximum(jnp.maximum(acc_e, acc_o) + bb2_ref[...], 0.0)
    m2 = m2.reshape(tb, 5, 2, 320)
    hp2 = jnp.maximum(m2[:, :, 0, :], m2[:, :, 1, :]).astype(jnp.bfloat16)

    acc = None
    for h in range(5):
        p = jnp.dot(hp2[:, h, :], f1_ref[h], preferred_element_type=jnp.float32)
        acc = p if acc is None else acc + p
    hfc = jnp.maximum(acc + fb1_ref[...], 0.0).astype(jnp.bfloat16)
    o_ref[...] = jnp.dot(hfc, f2_ref[...],
                         preferred_element_type=jnp.float32) + fb2_ref[...]


def _band_select(ow_count, w_count):
    s = np.zeros((5, 2, w_count, ow_count), np.float32)
    for kw in range(5):
        for p in range(2):
            for ow2 in range(ow_count):
                s[kw, p, 2 * ow2 + p + kw, ow2] = 1.0
    return s


def _build_tables(conv1_w, conv1_b, conv2_w, conv2_b,
                  fc1_w, fc1_b, fc2_w, fc2_b):
    f32 = jnp.float32
    bf16 = jnp.bfloat16
    w1 = conv1_w[:, 0].transpose(1, 2, 0).astype(f32)
    w2 = conv2_w.transpose(2, 3, 1, 0).astype(f32)

    s1 = jnp.asarray(_band_select(14, 32))
    b1 = jnp.einsum('akc,kpwm->apwmc', w1, s1)
    b1 = jnp.pad(b1, ((0, 0),) * 4 + ((0, 12),))
    b1 = b1.reshape(10, 32, 448).astype(bf16)

    s2 = jnp.asarray(_band_select(5, 14))
    t2 = jnp.einsum('akco,kpwm->apwcmo', w2, s2)
    t2 = jnp.pad(t2, ((0, 0), (0, 0), (0, 0), (0, 12), (0, 0), (0, 14)))
    b2 = t2.reshape(10, 448, 320).astype(bf16)

    t1 = fc1_w.astype(f32).reshape(320, 50, 5, 5).transpose(2, 3, 1, 0)
    t1 = jnp.pad(t1, ((0, 0), (0, 0), (0, 14), (0, 0)))
    f1 = jnp.pad(t1.reshape(5, 320, 320), ((0, 0), (0, 0), (0, 64))).astype(bf16)

    f2 = jnp.pad(fc2_w.astype(f32).T, ((0, 64), (0, 118))).astype(bf16)

    bb1 = jnp.tile(jnp.pad(conv1_b.astype(f32), (0, 12)), 14)[None]
    bb2 = jnp.tile(jnp.pad(conv2_b.astype(f32), (0, 14)), 5)[None]
    fb1 = jnp.pad(fc1_b.astype(f32), (0, 64))[None]
    fb2 = jnp.pad(fc2_b.astype(f32), (0, 118))[None]
    return b1, b2, f1, f2, bb1, bb2, fb1, fb2


def kernel(x, conv1_w, conv1_b, conv2_w, conv2_b, fc1_w, fc1_b, fc2_w, fc2_b):
    tables = _build_tables(conv1_w, conv1_b, conv2_w, conv2_b,
                           fc1_w, fc1_b, fc2_w, fc2_b)
    B = x.shape[0]
    xr = x.reshape(B, 28, 28).astype(jnp.float32)
    out = pl.pallas_call(
        _fused_kernel,
        out_shape=jax.ShapeDtypeStruct((B, 128), jnp.float32),
        grid=(B // _TB,),
        in_specs=[
            pl.BlockSpec((_TB, 28, 28), lambda i: (i, 0, 0)),
            pl.BlockSpec((10, 32, 448), lambda i: (0, 0, 0)),
            pl.BlockSpec((10, 448, 320), lambda i: (0, 0, 0)),
            pl.BlockSpec((5, 320, 384), lambda i: (0, 0, 0)),
            pl.BlockSpec((384, 128), lambda i: (0, 0)),
            pl.BlockSpec((1, 448), lambda i: (0, 0)),
            pl.BlockSpec((1, 320), lambda i: (0, 0)),
            pl.BlockSpec((1, 384), lambda i: (0, 0)),
            pl.BlockSpec((1, 128), lambda i: (0, 0)),
        ],
        out_specs=pl.BlockSpec((_TB, 128), lambda i: (i, 0)),
        compiler_params=pltpu.CompilerParams(dimension_semantics=("parallel",)),
    )(xr, *tables)
    return out[:, :10]


# speedup vs baseline: 27.4907x; 1.0417x over previous
import numpy as np

import jax
import jax.numpy as jnp
from jax.experimental import pallas as pl
from jax.experimental.pallas import tpu as pltpu

_TB = 64


def _fused_kernel(x_ref, b1_ref, b2_ref, f1_ref, f2_ref,
                  bb1_ref, bb2_ref, fb1_ref, fb2_ref, o_ref):
    tb = x_ref.shape[0]
    xp = jnp.pad(x_ref[...], ((0, 0), (2, 2), (2, 2))).astype(jnp.bfloat16)

    acc_e = acc_o = None
    for kh in range(5):
        a = xp[:, kh:kh + 28, :].reshape(tb * 28, 32)
        pe = jnp.dot(a, b1_ref[2 * kh], preferred_element_type=jnp.float32)
        po = jnp.dot(a, b1_ref[2 * kh + 1], preferred_element_type=jnp.float32)
        acc_e = pe if acc_e is None else acc_e + pe
        acc_o = po if acc_o is None else acc_o + po
    m = jnp.maximum(jnp.maximum(acc_e, acc_o) + bb1_ref[...], 0.0)
    m = m.reshape(tb, 14, 2, 448)
    hp = jnp.maximum(m[:, :, 0, :], m[:, :, 1, :]).astype(jnp.bfloat16)

    acc_e = acc_o = None
    for kh in range(5):
        a = hp[:, kh:kh + 10, :].reshape(tb * 10, 448)
        pe = jnp.dot(a, b2_ref[2 * kh], preferred_element_type=jnp.float32)
        po = jnp.dot(a, b2_ref[2 * kh + 1], preferred_element_type=jnp.float32)
        acc_e = pe if acc_e is None else acc_e + pe
        acc_o = po if acc_o is None else acc_o + po
    m2 = jnp.maximum(jnp.maximum(acc_e, acc_o) + bb2_ref[...], 0.0)
    m2 = m2.reshape(tb, 5, 2, 320)
    hp2 = jnp.maximum(m2[:, :, 0, :], m2[:, :, 1, :]).astype(jnp.bfloat16)

    acc = None
    for h in range(5):
        p = jnp.dot(hp2[:, h, :], f1_ref[h], preferred_element_type=jnp.float32)
        acc = p if acc is None else acc + p
    hfc = jnp.maximum(acc + fb1_ref[...], 0.0).astype(jnp.bfloat16)
    o_ref[...] = jnp.dot(hfc, f2_ref[...],
                         preferred_element_type=jnp.float32) + fb2_ref[...]


def _band_select(ow_count, w_count):
    s = np.zeros((5, 2, w_count, ow_count), np.float32)
    for kw in range(5):
        for p in range(2):
            for ow2 in range(ow_count):
                s[kw, p, 2 * ow2 + p + kw, ow2] = 1.0
    return s


def _build_tables(conv1_w, conv1_b, conv2_w, conv2_b,
                  fc1_w, fc1_b, fc2_w, fc2_b):
    f32 = jnp.float32
    bf16 = jnp.bfloat16
    w1 = conv1_w[:, 0].transpose(1, 2, 0).astype(f32)
    w2 = conv2_w.transpose(2, 3, 1, 0).astype(f32)

    s1 = jnp.asarray(_band_select(14, 32))
    b1 = jnp.einsum('akc,kpwm->apwmc', w1, s1)
    b1 = jnp.pad(b1, ((0, 0),) * 4 + ((0, 12),))
    b1 = b1.reshape(10, 32, 448).astype(bf16)

    s2 = jnp.asarray(_band_select(5, 14))
    t2 = jnp.einsum('akco,kpwm->apwcmo', w2, s2)
    t2 = jnp.pad(t2, ((0, 0), (0, 0), (0, 0), (0, 12), (0, 0), (0, 14)))
    b2 = t2.reshape(10, 448, 320).astype(bf16)

    t1 = fc1_w.astype(f32).reshape(320, 50, 5, 5).transpose(2, 3, 1, 0)
    t1 = jnp.pad(t1, ((0, 0), (0, 0), (0, 14), (0, 0)))
    f1 = jnp.pad(t1.reshape(5, 320, 320), ((0, 0), (0, 0), (0, 64))).astype(bf16)

    f2 = jnp.pad(fc2_w.astype(f32).T, ((0, 64), (0, 118))).astype(bf16)

    bb1 = jnp.tile(jnp.pad(conv1_b.astype(f32), (0, 12)), 14)[None]
    bb2 = jnp.tile(jnp.pad(conv2_b.astype(f32), (0, 14)), 5)[None]
    fb1 = jnp.pad(fc1_b.astype(f32), (0, 64))[None]
    fb2 = jnp.pad(fc2_b.astype(f32), (0, 118))[None]
    return b1, b2, f1, f2, bb1, bb2, fb1, fb2


def kernel(x, conv1_w, conv1_b, conv2_w, conv2_b, fc1_w, fc1_b, fc2_w, fc2_b):
    tables = _build_tables(conv1_w, conv1_b, conv2_w, conv2_b,
                           fc1_w, fc1_b, fc2_w, fc2_b)
    B = x.shape[0]
    xr = x.reshape(B, 28, 28).astype(jnp.float32)
    out = pl.pallas_call(
        _fused_kernel,
        out_shape=jax.ShapeDtypeStruct((B, 128), jnp.float32),
        grid=(B // _TB,),
        in_specs=[
            pl.BlockSpec((_TB, 28, 28), lambda i: (i, 0, 0)),
            pl.BlockSpec((10, 32, 448), lambda i: (0, 0, 0)),
            pl.BlockSpec((10, 448, 320), lambda i: (0, 0, 0)),
            pl.BlockSpec((5, 320, 384), lambda i: (0, 0, 0)),
            pl.BlockSpec((384, 128), lambda i: (0, 0)),
            pl.BlockSpec((1, 448), lambda i: (0, 0)),
            pl.BlockSpec((1, 320), lambda i: (0, 0)),
            pl.BlockSpec((1, 384), lambda i: (0, 0)),
            pl.BlockSpec((1, 128), lambda i: (0, 0)),
        ],
        out_specs=pl.BlockSpec((_TB, 128), lambda i: (i, 0)),
        compiler_params=pltpu.CompilerParams(dimension_semantics=("parallel",)),
    )(xr, *tables)
    return out[:, :10]


# speedup vs baseline: 39.3093x; 1.4299x over previous
import numpy as np

import jax
import jax.numpy as jnp
from jax.experimental import pallas as pl
from jax.experimental.pallas import tpu as pltpu

_TB = 64


def _fused_kernel(x_ref, b1_ref, b2_ref, f1_ref, f2_ref,
                  bb1_ref, bb2_ref, fb1_ref, fb2_ref, o_ref):
    tb = x_ref.shape[0]

    acc1 = [[None, None] for _ in range(4)]
    for r in range(4):
        for kh in range(5):
            src = (r + kh) % 4
            s = (r + kh) // 4
            a = x_ref[:, src, s:s + 7, :].reshape(tb * 7, 32)
            for p in range(2):
                d = jnp.dot(a, b1_ref[2 * kh + p],
                            preferred_element_type=jnp.float32)
                acc1[r][p] = d if acc1[r][p] is None else acc1[r][p] + d
    hpe = jnp.maximum(
        jnp.maximum(jnp.maximum(acc1[0][0], acc1[0][1]),
                    jnp.maximum(acc1[1][0], acc1[1][1])) + bb1_ref[...], 0.0
    ).astype(jnp.bfloat16).reshape(tb, 7, 448)
    hpo = jnp.maximum(
        jnp.maximum(jnp.maximum(acc1[2][0], acc1[2][1]),
                    jnp.maximum(acc1[3][0], acc1[3][1])) + bb1_ref[...], 0.0
    ).astype(jnp.bfloat16).reshape(tb, 7, 448)

    acc2 = [[None, None] for _ in range(2)]
    for q in range(2):
        for kh in range(5):
            src = hpe if (q + kh) % 2 == 0 else hpo
            t = (q + kh) // 2
            a = src[:, t:t + 5, :].reshape(tb * 5, 448)
            for p in range(2):
                d = jnp.dot(a, b2_ref[2 * kh + p],
                            preferred_element_type=jnp.float32)
                acc2[q][p] = d if acc2[q][p] is None else acc2[q][p] + d
    hp2 = jnp.maximum(
        jnp.maximum(jnp.maximum(acc2[0][0], acc2[0][1]),
                    jnp.maximum(acc2[1][0], acc2[1][1])) + bb2_ref[...], 0.0
    ).astype(jnp.bfloat16).reshape(tb, 5, 320)

    acc = None
    for h in range(5):
        d = jnp.dot(hp2[:, h, :], f1_ref[h], preferred_element_type=jnp.float32)
        acc = d if acc is None else acc + d
    hfc = jnp.maximum(acc + fb1_ref[...], 0.0).astype(jnp.bfloat16)
    o_ref[...] = jnp.dot(hfc, f2_ref[...],
                         preferred_element_type=jnp.float32) + fb2_ref[...]


def _band_select(ow_count, w_count):
    s = np.zeros((5, 2, w_count, ow_count), np.float32)
    for kw in range(5):
        for p in range(2):
            for ow2 in range(ow_count):
                s[kw, p, 2 * ow2 + p + kw, ow2] = 1.0
    return s


def _build_tables(conv1_w, conv1_b, conv2_w, conv2_b,
                  fc1_w, fc1_b, fc2_w, fc2_b):
    f32 = jnp.float32
    bf16 = jnp.bfloat16
    w1 = conv1_w[:, 0].transpose(1, 2, 0).astype(f32)
    w2 = conv2_w.transpose(2, 3, 1, 0).astype(f32)

    s1 = jnp.asarray(_band_select(14, 32))
    b1 = jnp.einsum('akc,kpwm->apwmc', w1, s1)
    b1 = jnp.pad(b1, ((0, 0),) * 4 + ((0, 12),))
    b1 = b1.reshape(10, 32, 448).astype(bf16)

    s2 = jnp.asarray(_band_select(5, 14))
    t2 = jnp.einsum('akco,kpwm->apwcmo', w2, s2)
    t2 = jnp.pad(t2, ((0, 0), (0, 0), (0, 0), (0, 12), (0, 0), (0, 14)))
    b2 = t2.reshape(10, 448, 320).astype(bf16)

    t1 = fc1_w.astype(f32).reshape(320, 50, 5, 5).transpose(2, 3, 1, 0)
    t1 = jnp.pad(t1, ((0, 0), (0, 0), (0, 14), (0, 0)))
    f1 = jnp.pad(t1.reshape(5, 320, 320), ((0, 0), (0, 0), (0, 64))).astype(bf16)

    f2 = jnp.pad(fc2_w.astype(f32).T, ((0, 64), (0, 118))).astype(bf16)

    bb1 = jnp.tile(jnp.pad(conv1_b.astype(f32), (0, 12)), 14)[None]
    bb2 = jnp.tile(jnp.pad(conv2_b.astype(f32), (0, 14)), 5)[None]
    fb1 = jnp.pad(fc1_b.astype(f32), (0, 64))[None]
    fb2 = jnp.pad(fc2_b.astype(f32), (0, 118))[None]
    return b1, b2, f1, f2, bb1, bb2, fb1, fb2


def kernel(x, conv1_w, conv1_b, conv2_w, conv2_b, fc1_w, fc1_b, fc2_w, fc2_b):
    tables = _build_tables(conv1_w, conv1_b, conv2_w, conv2_b,
                           fc1_w, fc1_b, fc2_w, fc2_b)
    B = x.shape[0]
    xp = jnp.pad(x.reshape(B, 28, 28).astype(jnp.float32),
                 ((0, 0), (2, 2), (2, 2)))
    x4 = xp.reshape(B, 8, 4, 32).transpose(0, 2, 1, 3).astype(jnp.bfloat16)
    out = pl.pallas_call(
        _fused_kernel,
        out_shape=jax.ShapeDtypeStruct((B, 128), jnp.float32),
        grid=(B // _TB,),
        in_specs=[
            pl.BlockSpec((_TB, 4, 8, 32), lambda i: (i, 0, 0, 0)),
            pl.BlockSpec((10, 32, 448), lambda i: (0, 0, 0)),
            pl.BlockSpec((10, 448, 320), lambda i: (0, 0, 0)),
            pl.BlockSpec((5, 320, 384), lambda i: (0, 0, 0)),
            pl.BlockSpec((384, 128), lambda i: (0, 0)),
            pl.BlockSpec((1, 448), lambda i: (0, 0)),
            pl.BlockSpec((1, 320), lambda i: (0, 0)),
            pl.BlockSpec((1, 384), lambda i: (0, 0)),
            pl.BlockSpec((1, 128), lambda i: (0, 0)),
        ],
        out_specs=pl.BlockSpec((_TB, 128), lambda i: (i, 0)),
        compiler_params=pltpu.CompilerParams(dimension_semantics=("parallel",)),
    )(x4, *tables)
    return out[:, :10]


# speedup vs baseline: 40.4412x; 1.0288x over previous
import numpy as np

import jax
import jax.numpy as jnp
from jax.experimental import pallas as pl
from jax.experimental.pallas import tpu as pltpu

_TB = 128


def _fused_kernel(x_ref, b1_ref, b2_ref, f1_ref, f2_ref,
                  bb1_ref, bb2_ref, fb1_ref, fb2_ref, o_ref):
    tb = x_ref.shape[0]

    acc1 = [[None, None] for _ in range(4)]
    for r in range(4):
        for kh in range(5):
            src = (r + kh) % 4
            s = (r + kh) // 4
            a = x_ref[:, src, s:s + 7, :].reshape(tb * 7, 32)
            for p in range(2):
                d = jnp.dot(a, b1_ref[2 * kh + p],
                            preferred_element_type=jnp.float32)
                acc1[r][p] = d if acc1[r][p] is None else acc1[r][p] + d
    hpe = jnp.maximum(
        jnp.maximum(jnp.maximum(acc1[0][0], acc1[0][1]),
                    jnp.maximum(acc1[1][0], acc1[1][1])) + bb1_ref[...], 0.0
    ).astype(jnp.bfloat16).reshape(tb, 7, 448)
    hpo = jnp.maximum(
        jnp.maximum(jnp.maximum(acc1[2][0], acc1[2][1]),
                    jnp.maximum(acc1[3][0], acc1[3][1])) + bb1_ref[...], 0.0
    ).astype(jnp.bfloat16).reshape(tb, 7, 448)

    acc2 = [[None, None] for _ in range(2)]
    for q in range(2):
        for kh in range(5):
            src = hpe if (q + kh) % 2 == 0 else hpo
            t = (q + kh) // 2
            a = src[:, t:t + 5, :].reshape(tb * 5, 448)
            for p in range(2):
                d = jnp.dot(a, b2_ref[2 * kh + p],
                            preferred_element_type=jnp.float32)
                acc2[q][p] = d if acc2[q][p] is None else acc2[q][p] + d
    hp2 = jnp.maximum(
        jnp.maximum(jnp.maximum(acc2[0][0], acc2[0][1]),
                    jnp.maximum(acc2[1][0], acc2[1][1])) + bb2_ref[...], 0.0
    ).astype(jnp.bfloat16).reshape(tb, 5, 320)

    acc = None
    for h in range(5):
        d = jnp.dot(hp2[:, h, :], f1_ref[h], preferred_element_type=jnp.float32)
        acc = d if acc is None else acc + d
    hfc = jnp.maximum(acc + fb1_ref[...], 0.0).astype(jnp.bfloat16)
    o_ref[...] = jnp.dot(hfc, f2_ref[...],
                         preferred_element_type=jnp.float32) + fb2_ref[...]


def _band_select(ow_count, w_count):
    s = np.zeros((5, 2, w_count, ow_count), np.float32)
    for kw in range(5):
        for p in range(2):
            for ow2 in range(ow_count):
                s[kw, p, 2 * ow2 + p + kw, ow2] = 1.0
    return s


def _build_tables(conv1_w, conv1_b, conv2_w, conv2_b,
                  fc1_w, fc1_b, fc2_w, fc2_b):
    f32 = jnp.float32
    bf16 = jnp.bfloat16
    w1 = conv1_w[:, 0].transpose(1, 2, 0).astype(f32)
    w2 = conv2_w.transpose(2, 3, 1, 0).astype(f32)

    s1 = jnp.asarray(_band_select(14, 32))
    b1 = jnp.einsum('akc,kpwm->apwmc', w1, s1)
    b1 = jnp.pad(b1, ((0, 0),) * 4 + ((0, 12),))
    b1 = b1.reshape(10, 32, 448).astype(bf16)

    s2 = jnp.asarray(_band_select(5, 14))
    t2 = jnp.einsum('akco,kpwm->apwcmo', w2, s2)
    t2 = jnp.pad(t2, ((0, 0), (0, 0), (0, 0), (0, 12), (0, 0), (0, 14)))
    b2 = t2.reshape(10, 448, 320).astype(bf16)

    t1 = fc1_w.astype(f32).reshape(320, 50, 5, 5).transpose(2, 3, 1, 0)
    t1 = jnp.pad(t1, ((0, 0), (0, 0), (0, 14), (0, 0)))
    f1 = jnp.pad(t1.reshape(5, 320, 320), ((0, 0), (0, 0), (0, 64))).astype(bf16)

    f2 = jnp.pad(fc2_w.astype(f32).T, ((0, 64), (0, 118))).astype(bf16)

    bb1 = jnp.tile(jnp.pad(conv1_b.astype(f32), (0, 12)), 14)[None]
    bb2 = jnp.tile(jnp.pad(conv2_b.astype(f32), (0, 14)), 5)[None]
    fb1 = jnp.pad(fc1_b.astype(f32), (0, 64))[None]
    fb2 = jnp.pad(fc2_b.astype(f32), (0, 118))[None]
    return b1, b2, f1, f2, bb1, bb2, fb1, fb2


def kernel(x, conv1_w, conv1_b, conv2_w, conv2_b, fc1_w, fc1_b, fc2_w, fc2_b):
    tables = _build_tables(conv1_w, conv1_b, conv2_w, conv2_b,
                           fc1_w, fc1_b, fc2_w, fc2_b)
    B = x.shape[0]
    xp = jnp.pad(x.reshape(B, 28, 28).astype(jnp.float32),
                 ((0, 0), (2, 2), (2, 2)))
    x4 = xp.reshape(B, 8, 4, 32).transpose(0, 2, 1, 3).astype(jnp.bfloat16)
    out = pl.pallas_call(
        _fused_kernel,
        out_shape=jax.ShapeDtypeStruct((B, 128), jnp.float32),
        grid=(B // _TB,),
        in_specs=[
            pl.BlockSpec((_TB, 4, 8, 32), lambda i: (i, 0, 0, 0)),
            pl.BlockSpec((10, 32, 448), lambda i: (0, 0, 0)),
            pl.BlockSpec((10, 448, 320), lambda i: (0, 0, 0)),
            pl.BlockSpec((5, 320, 384), lambda i: (0, 0, 0)),
            pl.BlockSpec((384, 128), lambda i: (0, 0)),
            pl.BlockSpec((1, 448), lambda i: (0, 0)),
            pl.BlockSpec((1, 320), lambda i: (0, 0)),
            pl.BlockSpec((1, 384), lambda i: (0, 0)),
            pl.BlockSpec((1, 128), lambda i: (0, 0)),
        ],
        out_specs=pl.BlockSpec((_TB, 128), lambda i: (i, 0)),
        compiler_params=pltpu.CompilerParams(dimension_semantics=("parallel",)),
    )(x4, *tables)
    return out[:, :10]
